# Initial kernel scaffold; baseline (speedup 1.0000x reference)
import jax
import jax.numpy as jnp
from jax.experimental import pallas as pl


def kernel(logits, norms, labels):
    raise NotImplementedError("write your pallas kernel here")


# speedup vs baseline: 1.1199x; 1.1199x over previous
import jax
import jax.numpy as jnp
from jax.experimental import pallas as pl

_S = 64.0
_M = 0.4

_BLOCK_C = 2048


def _body(labels_ref, x_ref, o_ref):
    j = pl.program_id(0)
    col0 = j * _BLOCK_C
    b, bc = x_ref.shape
    cols = col0 + jax.lax.broadcasted_iota(jnp.int32, (b, bc), 1)
    lab = labels_ref[...]
    x = x_ref[...]
    o_ref[...] = (x - jnp.where(cols == lab, _M, 0.0)) * _S


def kernel(logits, norms, labels):
    del norms
    b, c = logits.shape
    labels2d = labels.astype(jnp.int32).reshape(b, 1)
    grid = (pl.cdiv(c, _BLOCK_C),)
    return pl.pallas_call(
        _body,
        grid=grid,
        in_specs=[
            pl.BlockSpec((b, 1), lambda j: (0, 0)),
            pl.BlockSpec((b, _BLOCK_C), lambda j: (0, j)),
        ],
        out_specs=pl.BlockSpec((b, _BLOCK_C), lambda j: (0, j)),
        out_shape=jax.ShapeDtypeStruct((b, c), jnp.float32),
    )(labels2d, logits)
